# baseline (speedup 1.0000x reference)
import functools

import jax
import jax.numpy as jnp
from jax import lax
from jax.experimental import pallas as pl
from jax.experimental.pallas import tpu as pltpu
from jax.experimental.pallas import tpu_sc as plsc

VOCAB = 1_000_000
EMB = 64
CTX = 2
B = 1024
L = 20

NC = 2
NS = 16
NW = NC * NS
ROWS_PER_W = B // NW
IDX_PER_W = ROWS_PER_W * L
CHUNK = 128
NCHUNK = IDX_PER_W // CHUNK
LANES = 16
KCOL = EMB // LANES


def _cbow_body(x_hbm, w_hbm, out_hbm, idx_v, rows_v, out_v, w0_v, sem):
    wid = lax.axis_index("s") * NC + lax.axis_index("c")

    pltpu.sync_copy(x_hbm.at[wid], idx_v)
    pltpu.sync_copy(w_hbm.at[pl.ds(0, 1)], w0_v)

    copies = [
        pltpu.async_copy(
            w_hbm.at[idx_v.at[c]],
            rows_v.at[pl.ds(c * CHUNK, CHUNK)],
            sem,
        )
        for c in range(NCHUNK)
    ]
    for cp in copies:
        cp.wait()

    def body(b, carry):
        base = b * L
        for k in range(KCOL):
            col = pl.ds(k * LANES, LANES)
            w0 = w0_v[0, col]
            r0 = rows_v[base, col]
            r1 = rows_v[base + 1, col]
            s = r0 + r1
            for j in range(2, L - 2):
                s = s + rows_v[base + j, col]
            r18 = rows_v[base + L - 2, col]
            r19 = rows_v[base + L - 1, col]
            s = s + r18 + r19
            t = s + w0
            o0 = t - r19
            o1 = o0 + w0 - r18
            o2 = t - r0
            o3 = o2 + w0 - r1
            out_v[b, 0, col] = o0
            out_v[b, 1, col] = o1
            out_v[b, 2, col] = o2
            out_v[b, 3, col] = o3
        return carry

    lax.fori_loop(0, ROWS_PER_W, body, 0)

    pltpu.sync_copy(out_v, out_hbm.at[pl.ds(wid * ROWS_PER_W, ROWS_PER_W)])


def kernel(x, W):
    x3 = x.reshape(NW, NCHUNK, CHUNK).astype(jnp.int32)
    mesh = plsc.VectorSubcoreMesh(core_axis_name="c", subcore_axis_name="s")
    f = functools.partial(
        pl.kernel,
        mesh=mesh,
        out_type=jax.ShapeDtypeStruct((B, 2 * CTX, EMB), jnp.float32),
        scratch_types=[
            pltpu.VMEM((NCHUNK, CHUNK), jnp.int32),
            pltpu.VMEM((IDX_PER_W, EMB), jnp.float32),
            pltpu.VMEM((ROWS_PER_W, 2 * CTX, EMB), jnp.float32),
            pltpu.VMEM((1, EMB), jnp.float32),
            pltpu.SemaphoreType.DMA,
        ],
        compiler_params=pltpu.CompilerParams(use_tc_tiling_on_sc=False),
    )(_cbow_body)
    return f(x3, W)
